# baseline (speedup 1.0000x reference)
import jax
import jax.numpy as jnp
from jax import lax
from jax.experimental import pallas as pl
from jax.experimental.pallas import tpu as pltpu
from jax.experimental.pallas import tpu_sc as plsc

N = 1000000
D = 8
K = 1024
NW = 32
R = 2048
NFULL = N // R
TAIL = N - NFULL * R
TASKS = -(-NFULL // NW)
TAIL_W = NFULL % NW
DWORDS = D * D // 2
DSTRIDE = DWORDS + 1
MWORDS = D // 2
MSTRIDE = MWORDS + 1
MASK_HI = -65536


def _process_rows(zv, outv, devs_v, means_v, part_v, ngroups):
    lane = lax.iota(jnp.int32, 16)
    lane_z = lane * (D + 1)
    lane_o = lane * D

    @plsc.parallel_loop(0, ngroups, unroll=8)
    def group(g):
        zoff = g * (16 * (D + 1)) + lane_z
        u = plsc.load_gather(zv, [zoff])
        x = [plsc.load_gather(zv, [zoff + (1 + j)]) for j in range(D)]

        pos = jnp.zeros(16, jnp.int32)
        b = K // 2
        while b >= 1:
            pm = plsc.load_gather(part_v, [pos + (b - 1)])
            pos = jnp.where(pm <= u, pos + b, pos)
            b //= 2
        idx = pos

        mbase = (idx << 2) + idx
        dbase = (idx << 5) + idx
        obase = g * (16 * D) + lane_o
        for i in range(D):
            mw = plsc.load_gather(means_v, [mbase + (i // 2)])
            if i % 2 == 0:
                acc = plsc.bitcast(mw & MASK_HI, jnp.float32)
            else:
                acc = plsc.bitcast(mw << 16, jnp.float32)
            for t in range(D // 2):
                w = plsc.load_gather(devs_v, [dbase + (i * (D // 2) + t)])
                d0 = plsc.bitcast(w & MASK_HI, jnp.float32)
                d1 = plsc.bitcast(w << 16, jnp.float32)
                acc = acc + d0 * x[2 * t] + d1 * x[2 * t + 1]
            plsc.store_scatter(outv, [obase + i], acc)


ZLEN = R * (D + 1)
OLEN = R * D


def _body(z_hbm, means_hbm, devs_hbm, part_hbm, out_hbm,
          devs_v, means_v, part_v, zvs, outvs, zsems, osems):
    pltpu.sync_copy(devs_hbm, devs_v)
    pltpu.sync_copy(means_hbm, means_v)
    pltpu.sync_copy(part_hbm, part_v)

    wid = lax.axis_index("s") * 2 + lax.axis_index("c")

    def valid(t):
        return (wid + NW * t) < NFULL

    def zslice(t):
        return z_hbm.at[pl.ds(pl.multiple_of((wid + NW * t) * ZLEN, 8), ZLEN)]

    def oslice(t):
        return out_hbm.at[pl.ds(pl.multiple_of((wid + NW * t) * OLEN, 8), OLEN)]

    @pl.when(valid(0))
    def _():
        pltpu.async_copy(zslice(0), zvs[0], zsems[0])

    def pipelined(t, a):
        zvA, outvA, zvB = zvs[a], outvs[a], zvs[1 - a]

        @pl.when(valid(t + 1))
        def _():
            pltpu.async_copy(zslice(t + 1), zvB, zsems[1 - a])

        @pl.when(valid(t))
        def _():
            pltpu.make_async_copy(zslice(t), zvA, zsems[a]).wait()

            @pl.when(t >= 2)
            def _():
                pltpu.make_async_copy(outvA, oslice(t - 2), osems[a]).wait()

            _process_rows(zvA, outvA, devs_v, means_v, part_v, R // 16)
            pltpu.async_copy(outvA, oslice(t), osems[a])

    def task(t, _):
        even = (t & 1) == 0

        @pl.when(even)
        def _():
            pipelined(t, 0)

        @pl.when(jnp.logical_not(even))
        def _():
            pipelined(t, 1)

        return 0

    lax.fori_loop(0, TASKS, task, 0)

    for tt in range(TASKS):
        @pl.when(valid(tt) & jnp.logical_not(valid(tt + 2)))
        def _(tt=tt):
            pltpu.make_async_copy(outvs[tt % 2], oslice(tt),
                                  osems[tt % 2]).wait()

    @pl.when(wid == TAIL_W)
    def _():
        zoff = pl.multiple_of(NFULL * (R * (D + 1)), 8)
        ooff = pl.multiple_of(NFULL * (R * D), 8)
        pltpu.sync_copy(z_hbm.at[pl.ds(zoff, TAIL * (D + 1))],
                        zvs[0].at[pl.ds(0, TAIL * (D + 1))])
        _process_rows(zvs[0], outvs[0], devs_v, means_v, part_v, TAIL // 16)
        pltpu.sync_copy(outvs[0].at[pl.ds(0, TAIL * D)],
                        out_hbm.at[pl.ds(ooff, TAIL * D)])


@jax.jit
def _run(zf, meansp, devsp, part):
    mesh = plsc.VectorSubcoreMesh(core_axis_name="c", subcore_axis_name="s")
    return pl.kernel(
        _body,
        mesh=mesh,
        compiler_params=pltpu.CompilerParams(needs_layout_passes=False,
                                             use_tc_tiling_on_sc=False),
        out_type=jax.ShapeDtypeStruct((N * D,), jnp.float32),
        scratch_types=[
            pltpu.VMEM((K * DSTRIDE,), jnp.int32),
            pltpu.VMEM((K * MSTRIDE,), jnp.int32),
            pltpu.VMEM((K,), jnp.float32),
            [pltpu.VMEM((R * (D + 1),), jnp.float32) for _ in range(2)],
            [pltpu.VMEM((R * D,), jnp.float32) for _ in range(2)],
            [pltpu.SemaphoreType.DMA for _ in range(2)],
            [pltpu.SemaphoreType.DMA for _ in range(2)],
        ],
    )(zf, meansp, devsp, part)


def _pack_bf16_pairs(a, nwords, stride):
    pair = a.reshape(K, nwords, 2).astype(jnp.bfloat16)
    bits = pair.view(jnp.uint16).astype(jnp.uint32)
    words = (bits[..., 0] << 16 | bits[..., 1]).astype(jnp.int32)
    return jnp.pad(words, ((0, 0), (0, stride - nwords))).reshape(-1)


def kernel(z, means, devs, mix_partition):
    meansp = _pack_bf16_pairs(means, MWORDS, MSTRIDE)
    devsp = _pack_bf16_pairs(devs.reshape(K, D * D), DWORDS, DSTRIDE)
    out = _run(z.reshape(-1), meansp, devsp, mix_partition)
    return out.reshape(N, D)
